# baseline (speedup 1.0000x reference)
import functools

import jax
import jax.numpy as jnp
from jax import lax
from jax.experimental import pallas as pl
from jax.experimental.pallas import tpu as pltpu
from jax.experimental.pallas import tpu_sc as plsc

LANES = 16
NBUF = 4
PF = 2


@functools.lru_cache(maxsize=None)
def _build(batch, seqlen, embed, vocab):
    info = plsc.get_sparse_core_info()
    nc, ns = info.num_cores, info.num_subcores
    nw = nc * ns

    assert batch % (nw * 128) == 0
    bblk = batch // nw
    assert bblk == 128
    assert embed % LANES == 0
    esub = embed // 8
    nchunk = seqlen
    assert nchunk % NBUF == 0 and nchunk >= 2 * NBUF

    mesh = plsc.VectorSubcoreMesh(core_axis_name="c", subcore_axis_name="s")

    @functools.partial(
        pl.kernel,
        mesh=mesh,
        compiler_params=pltpu.CompilerParams(
            use_tc_tiling_on_sc=False, needs_layout_passes=False
        ),
        out_type=jax.ShapeDtypeStruct(
            (seqlen, esub, nw, 8, bblk), jnp.float32
        ),
        scratch_types=[
            pltpu.VMEM((seqlen, bblk), jnp.int32),
            pltpu.VMEM((seqlen, embed), jnp.float32),
        ]
        + [pltpu.VMEM((bblk, embed), jnp.float32) for _ in range(NBUF)]
        + [pltpu.VMEM((embed, bblk), jnp.float32) for _ in range(NBUF)]
        + [pltpu.SemaphoreType.DMA for _ in range(2 * NBUF)],
    )
    def emb(idx_hbm, table_hbm, pos_hbm, out_hbm, idx_v, pos_v, *bufsem):
        gbufs = bufsem[:NBUF]
        tbufs = bufsem[NBUF : 2 * NBUF]
        gsems = bufsem[2 * NBUF : 3 * NBUF]
        wsems = bufsem[3 * NBUF :]
        wid = lax.axis_index("s") * nc + lax.axis_index("c")
        pltpu.sync_copy(idx_hbm.at[:, pl.ds(wid * bblk, bblk)], idx_v)
        pltpu.sync_copy(pos_hbm, pos_v)

        iota = lax.iota(jnp.int32, LANES)

        def gather_desc(s, b):
            return pltpu.make_async_copy(
                table_hbm.at[idx_v.at[s, :]], gbufs[b], gsems[b]
            )

        def write_descs(s, b):
            return [
                pltpu.make_async_copy(
                    tbufs[b].at[pl.ds(et * 8, 8), :],
                    out_hbm.at[s, et, wid],
                    wsems[b],
                )
                for et in range(esub)
            ]

        def compute(s, b):
            def e_body(e, carry):
                e_sp = jnp.full((LANES,), 0, jnp.int32) + e
                s_sp = jnp.full((LANES,), 0, jnp.int32) + s
                ps = plsc.load_gather(pos_v, [s_sp, e_sp])
                for blk in range(bblk // LANES):
                    g = plsc.load_gather(gbufs[b], [iota + blk * LANES, e_sp])
                    tbufs[b][e, pl.ds(blk * LANES, LANES)] = g + ps
                return carry

            lax.fori_loop(0, embed, e_body, 0)

        def step(s, b, *, first, issue):
            gather_desc(s, b).wait()
            if issue:
                gather_desc(s + PF, (b + PF) % NBUF).start()
            if not first:
                for d in write_descs(s - NBUF, b):
                    d.wait()
            compute(s, b)
            for d in write_descs(s, b):
                d.start()

        for s in range(PF):
            gather_desc(s, s % NBUF).start()
        for s in range(NBUF):
            step(s, s % NBUF, first=True, issue=True)

        def group_body(grp, carry):
            for j in range(NBUF):
                s = NBUF + grp * NBUF + j
                step(s, j, first=False, issue=True)
            return carry

        lax.fori_loop(0, (nchunk - 2 * NBUF) // NBUF, group_body, 0)

        for s in range(nchunk - NBUF, nchunk):
            step(s, s % NBUF, first=False, issue=(s + PF < nchunk))
        for s in range(nchunk - NBUF, nchunk):
            for d in write_descs(s, s % NBUF):
                d.wait()

    return emb


def kernel(inputs, index_table, pos_table):
    batch, seqlen = inputs.shape
    vocab, embed = index_table.shape
    nw = batch // 128
    emb = _build(batch, seqlen, embed, vocab)
    out5d = emb(
        inputs.T.astype(jnp.int32),
        index_table.astype(jnp.float32),
        pos_table.astype(jnp.float32),
    )
    return out5d.transpose(2, 4, 0, 1, 3).reshape(batch, seqlen, embed)


# speedup vs baseline: 1.7527x; 1.7527x over previous
import functools

import jax
import jax.numpy as jnp
from jax import lax
from jax.experimental import pallas as pl
from jax.experimental.pallas import tpu as pltpu
from jax.experimental.pallas import tpu_sc as plsc

LANES = 16
NBUF = 4
PF = 2


@functools.lru_cache(maxsize=None)
def _build(batch, seqlen, embed, vocab):
    info = plsc.get_sparse_core_info()
    nc, ns = info.num_cores, info.num_subcores
    nw = nc * ns

    assert batch % (nw * 128) == 0
    bblk = batch // nw
    assert bblk == 128
    assert embed % LANES == 0
    esub = embed // 8
    evec = embed // LANES
    nchunk = seqlen
    assert nchunk % NBUF == 0 and nchunk >= 2 * NBUF

    mesh = plsc.VectorSubcoreMesh(core_axis_name="c", subcore_axis_name="s")

    @functools.partial(
        pl.kernel,
        mesh=mesh,
        compiler_params=pltpu.CompilerParams(
            use_tc_tiling_on_sc=False, needs_layout_passes=False
        ),
        out_type=jax.ShapeDtypeStruct(
            (seqlen, esub, nw, 8, bblk), jnp.float32
        ),
        scratch_types=[
            pltpu.VMEM((seqlen, bblk), jnp.int32),
            pltpu.VMEM((seqlen, embed), jnp.float32),
        ]
        + [pltpu.VMEM((bblk, embed), jnp.float32) for _ in range(NBUF)]
        + [pltpu.VMEM((embed, bblk + 1), jnp.float32) for _ in range(NBUF)]
        + [pltpu.SemaphoreType.DMA for _ in range(2 * NBUF)],
    )
    def emb(idx_hbm, table_hbm, pos_hbm, out_hbm, idx_v, pos_v, *bufsem):
        gbufs = bufsem[:NBUF]
        tbufs = bufsem[NBUF : 2 * NBUF]
        gsems = bufsem[2 * NBUF : 3 * NBUF]
        wsems = bufsem[3 * NBUF :]
        wid = lax.axis_index("s") * nc + lax.axis_index("c")
        pltpu.sync_copy(idx_hbm.at[:, pl.ds(wid * bblk, bblk)], idx_v)
        pltpu.sync_copy(pos_hbm, pos_v)

        iota = lax.iota(jnp.int32, LANES)

        def gather_desc(s, b):
            return pltpu.make_async_copy(
                table_hbm.at[idx_v.at[s, :]], gbufs[b], gsems[b]
            )

        def write_descs(s, b):
            return [
                pltpu.make_async_copy(
                    tbufs[b].at[pl.ds(et * 8, 8), pl.ds(0, bblk)],
                    out_hbm.at[s, et, wid],
                    wsems[b],
                )
                for et in range(esub)
            ]

        def compute(s, b):
            pos4 = [pos_v[s, pl.ds(c * LANES, LANES)] for c in range(evec)]
            rows = [iota + c * LANES for c in range(evec)]

            def b_body(bl, carry):
                col = jnp.full((LANES,), 0, jnp.int32) + bl
                for c in range(evec):
                    g = gbufs[b][bl, pl.ds(c * LANES, LANES)] + pos4[c]
                    plsc.store_scatter(tbufs[b], [rows[c], col], g)
                return carry

            lax.fori_loop(0, bblk, b_body, 0, unroll=4)

        def step(s, b, *, first, issue):
            gather_desc(s, b).wait()
            if issue:
                gather_desc(s + PF, (b + PF) % NBUF).start()
            if not first:
                for d in write_descs(s - NBUF, b):
                    d.wait()
            compute(s, b)
            for d in write_descs(s, b):
                d.start()

        for s in range(PF):
            gather_desc(s, s % NBUF).start()
        for s in range(NBUF):
            step(s, s % NBUF, first=True, issue=True)

        def group_body(grp, carry):
            for j in range(NBUF):
                s = NBUF + grp * NBUF + j
                step(s, j, first=False, issue=True)
            return carry

        lax.fori_loop(0, (nchunk - 2 * NBUF) // NBUF, group_body, 0)

        for s in range(nchunk - NBUF, nchunk):
            step(s, s % NBUF, first=False, issue=(s + PF < nchunk))
        for s in range(nchunk - NBUF, nchunk):
            for d in write_descs(s, s % NBUF):
                d.wait()

    return emb


def kernel(inputs, index_table, pos_table):
    batch, seqlen = inputs.shape
    vocab, embed = index_table.shape
    nw = batch // 128
    emb = _build(batch, seqlen, embed, vocab)
    out5d = emb(
        inputs.T.astype(jnp.int32),
        index_table.astype(jnp.float32),
        pos_table.astype(jnp.float32),
    )
    return out5d.transpose(2, 4, 0, 1, 3).reshape(batch, seqlen, embed)


# speedup vs baseline: 2.3841x; 1.3603x over previous
import functools

import jax
import jax.numpy as jnp
from jax import lax
from jax.experimental import pallas as pl
from jax.experimental.pallas import tpu as pltpu
from jax.experimental.pallas import tpu_sc as plsc

LANES = 16
NBUF = 4
PF = 2


@functools.lru_cache(maxsize=None)
def _build(batch, seqlen, embed, vocab):
    info = plsc.get_sparse_core_info()
    nc, ns = info.num_cores, info.num_subcores
    nw = nc * ns

    assert batch % (nw * 128) == 0
    bblk = batch // nw
    assert bblk == 128
    assert embed % LANES == 0
    esub = embed // 8
    evec = embed // LANES
    nchunk = seqlen
    assert nchunk % NBUF == 0 and nchunk >= 2 * NBUF

    mesh = plsc.VectorSubcoreMesh(core_axis_name="c", subcore_axis_name="s")

    @functools.partial(
        pl.kernel,
        mesh=mesh,
        compiler_params=pltpu.CompilerParams(
            use_tc_tiling_on_sc=False, needs_layout_passes=False
        ),
        out_type=jax.ShapeDtypeStruct(
            (seqlen, esub, nw, 8, bblk), jnp.float32
        ),
        scratch_types=[
            pltpu.VMEM((seqlen, bblk), jnp.int32),
            pltpu.VMEM((seqlen, embed), jnp.float32),
        ]
        + [pltpu.VMEM((bblk, embed), jnp.float32) for _ in range(NBUF)]
        + [pltpu.VMEM((embed, bblk + 1), jnp.float32) for _ in range(NBUF)]
        + [pltpu.SemaphoreType.DMA for _ in range(2 * NBUF)],
    )
    def emb(idx_hbm, table_hbm, pos_hbm, out_hbm, idx_v, pos_v, *bufsem):
        gbufs = bufsem[:NBUF]
        tbufs = bufsem[NBUF : 2 * NBUF]
        gsems = bufsem[2 * NBUF : 3 * NBUF]
        wsems = bufsem[3 * NBUF :]
        wid = lax.axis_index("s") * nc + lax.axis_index("c")
        pltpu.sync_copy(idx_hbm.at[:, pl.ds(wid * bblk, bblk)], idx_v)
        pltpu.sync_copy(pos_hbm, pos_v)

        iota = lax.iota(jnp.int32, LANES)

        def gather_desc(s, b):
            return pltpu.make_async_copy(
                table_hbm.at[idx_v.at[s, :]], gbufs[b], gsems[b]
            )

        def write_descs(s, b):
            return [
                pltpu.make_async_copy(
                    tbufs[b].at[pl.ds(et * 8, 8), pl.ds(0, bblk)],
                    out_hbm.at[s, et, wid],
                    wsems[b],
                )
                for et in range(esub)
            ]

        def compute(s, b):
            pos4 = [pos_v[s, pl.ds(c * LANES, LANES)] for c in range(evec)]
            rows = [iota + c * LANES for c in range(evec)]

            @plsc.parallel_loop(0, bblk, unroll=8)
            def b_body(bl):
                col = jnp.full((LANES,), 0, jnp.int32) + bl
                for c in range(evec):
                    g = gbufs[b][bl, pl.ds(c * LANES, LANES)] + pos4[c]
                    plsc.store_scatter(tbufs[b], [rows[c], col], g)

        def step(s, b, *, first, issue):
            gather_desc(s, b).wait()
            if issue:
                gather_desc(s + PF, (b + PF) % NBUF).start()
            if not first:
                for d in write_descs(s - NBUF, b):
                    d.wait()
            compute(s, b)
            for d in write_descs(s, b):
                d.start()

        for s in range(PF):
            gather_desc(s, s % NBUF).start()
        for s in range(NBUF):
            step(s, s % NBUF, first=True, issue=True)

        def group_body(grp, carry):
            for j in range(NBUF):
                s = NBUF + grp * NBUF + j
                step(s, j, first=False, issue=True)
            return carry

        lax.fori_loop(0, (nchunk - 2 * NBUF) // NBUF, group_body, 0)

        for s in range(nchunk - NBUF, nchunk):
            step(s, s % NBUF, first=False, issue=(s + PF < nchunk))
        for s in range(nchunk - NBUF, nchunk):
            for d in write_descs(s, s % NBUF):
                d.wait()

    return emb


def kernel(inputs, index_table, pos_table):
    batch, seqlen = inputs.shape
    vocab, embed = index_table.shape
    nw = batch // 128
    emb = _build(batch, seqlen, embed, vocab)
    out5d = emb(
        inputs.T.astype(jnp.int32),
        index_table.astype(jnp.float32),
        pos_table.astype(jnp.float32),
    )
    return out5d.transpose(2, 4, 0, 1, 3).reshape(batch, seqlen, embed)
